# baseline (speedup 1.0000x reference)
import functools

import jax
import jax.numpy as jnp
import numpy as np
from jax import lax
from jax.experimental import pallas as pl
from jax.experimental.pallas import tpu as pltpu
from jax.experimental.pallas import tpu_sc as plsc

_SHAPES = [(128, 128), (64, 64), (32, 32), (16, 16)]
_B = 2
_D = 256
_NLAYERS = 3
_NH = 8
_NL = 4
_NP = 4
_DH = _D // _NH
_LQ = sum(h * w for h, w in _SHAPES)
_CHUNK = 1280
_NBLK = _LQ // _CHUNK
_NQ = _B * _LQ
_NWORK = 32
_QPW = _NQ // _NWORK

_lane = np.arange(128)
_h_of = _lane // 16
_l_of = (_lane // 4) % 4
_W_LVL = np.array([w for h, w in _SHAPES], np.float32)
_H_LVL = np.array([h for h, w in _SHAPES], np.float32)
_BASE_LVL = np.cumsum([0] + [h * w for h, w in _SHAPES])[:4]

_SW = _W_LVL[_l_of].reshape(1, 128)
_SH = _H_LVL[_l_of].reshape(1, 128)
_BASE = _BASE_LVL[_l_of].astype(np.int32).reshape(1, 128)
_HL = _h_of.astype(np.int32).reshape(1, 128)

_G = np.kron(np.eye(8, dtype=np.float32), np.ones((16, 16), np.float32))

_PERM = np.empty(256, np.int32)
for _c in range(2):
    for _hh in range(8):
        for _ll in range(4):
            for _pp in range(4):
                _PERM[_c * 128 + _hh * 16 + _ll * 4 + _pp] = ((_hh * 4 + _ll) * 4 + _pp) * 2 + _c

_PERM_V = np.empty(256, np.int32)
for _hh in range(8):
    for _k in range(16):
        _PERM_V[_hh * 32 + 2 * _k] = _hh * 32 + _k
        _PERM_V[_hh * 32 + 2 * _k + 1] = _hh * 32 + 16 + _k


def _ref_points_np():
    xs, ys = [], []
    for h, w in _SHAPES:
        ry = (np.arange(h, dtype=np.float32) + 0.5) / h
        rx = (np.arange(w, dtype=np.float32) + 0.5) / w
        gy, gx = np.meshgrid(ry, rx, indexing="ij")
        xs.append(gx.reshape(-1))
        ys.append(gy.reshape(-1))
    return np.concatenate(xs), np.concatenate(ys)


_REFX, _REFY = _ref_points_np()
_REFX = _REFX.reshape(_LQ, 1)
_REFY = _REFY.reshape(_LQ, 1)


def _ka_body(x_ref, qp_ref, refx_ref, refy_ref, wso_ref, bso_ref, waw_ref,
             baw_ref, wv_ref, bv_ref, sw_ref, sh_ref, base_ref, hl_ref, g_ref,
             val_ref, idx_ref, wgt_ref):
    b = pl.program_id(0)
    x = x_ref[0]
    q = x + qp_ref[0]
    off = jnp.dot(q, wso_ref[...], preferred_element_type=jnp.float32) + bso_ref[...]
    logits = jnp.dot(q, waw_ref[...], preferred_element_type=jnp.float32) + baw_ref[...]
    m = jnp.max(logits, axis=-1, keepdims=True)
    e = jnp.exp(logits - m)
    gs = lax.dot(e, g_ref[...], precision=lax.Precision.HIGHEST)
    aw = e / gs
    val_ref[0] = (jnp.dot(x, wv_ref[...], preferred_element_type=jnp.float32)
                  + bv_ref[...]).astype(jnp.bfloat16)

    offx = off[:, :128]
    offy = off[:, 128:]
    sw = sw_ref[...]
    sh = sh_ref[...]
    xi = refx_ref[...] * sw + offx - 0.5
    yi = refy_ref[...] * sh + offy - 0.5
    x0 = jnp.floor(xi)
    y0 = jnp.floor(yi)
    fx = xi - x0
    fy = yi - y0
    x0i = x0.astype(jnp.int32)
    y0i = y0.astype(jnp.int32)
    swi = sw.astype(jnp.int32)
    shi = sh.astype(jnp.int32)
    rowbase = b * _LQ + base_ref[...]
    hl = hl_ref[...]

    outs_i = []
    outs_w = []
    for dy in (0, 1):
        wy = (1.0 - fy) if dy == 0 else fy
        yc = y0i + dy
        vy = (yc >= 0) & (yc <= shi - 1)
        cy = jnp.clip(yc, 0, shi - 1)
        for dx in (0, 1):
            wx = (1.0 - fx) if dx == 0 else fx
            xc = x0i + dx
            vx = (xc >= 0) & (xc <= swi - 1)
            cx = jnp.clip(xc, 0, swi - 1)
            wc = wx * wy * (vx & vy).astype(jnp.float32) * aw
            row = (rowbase + cy * swi + cx) * _NH + hl
            outs_i.append(row)
            outs_w.append(wc)
    idx_ref[0] = jnp.concatenate(outs_i, axis=-1)
    wgt_ref[0] = jnp.concatenate(outs_w, axis=-1)


def _kernel_a(x, qpos, refx, refy, wso, bso, waw, baw, wv, bv, interpret=False):
    c1 = lambda i, j: (0, 0)
    return pl.pallas_call(
        _ka_body,
        grid=(_B, _NBLK),
        in_specs=[
            pl.BlockSpec((1, _CHUNK, _D), lambda i, j: (i, j, 0)),
            pl.BlockSpec((1, _CHUNK, _D), lambda i, j: (i, j, 0)),
            pl.BlockSpec((_CHUNK, 1), lambda i, j: (j, 0)),
            pl.BlockSpec((_CHUNK, 1), lambda i, j: (j, 0)),
            pl.BlockSpec((_D, 256), c1),
            pl.BlockSpec((1, 256), c1),
            pl.BlockSpec((_D, 128), c1),
            pl.BlockSpec((1, 128), c1),
            pl.BlockSpec((_D, _D), c1),
            pl.BlockSpec((1, _D), c1),
            pl.BlockSpec((1, 128), c1),
            pl.BlockSpec((1, 128), c1),
            pl.BlockSpec((1, 128), c1),
            pl.BlockSpec((1, 128), c1),
            pl.BlockSpec((128, 128), c1),
        ],
        out_specs=[
            pl.BlockSpec((1, _CHUNK, _D), lambda i, j: (i, j, 0)),
            pl.BlockSpec((1, _CHUNK, 512), lambda i, j: (i, j, 0)),
            pl.BlockSpec((1, _CHUNK, 512), lambda i, j: (i, j, 0)),
        ],
        out_shape=[
            jax.ShapeDtypeStruct((_B, _LQ, _D), jnp.bfloat16),
            jax.ShapeDtypeStruct((_B, _LQ, 512), jnp.int32),
            jax.ShapeDtypeStruct((_B, _LQ, 512), jnp.float32),
        ],
        interpret=interpret,
    )(x, qpos, refx, refy, wso, bso, waw, baw, wv, bv,
      jnp.asarray(_SW), jnp.asarray(_SH), jnp.asarray(_BASE), jnp.asarray(_HL),
      jnp.asarray(_G))


def _bcast_lane(v, j):
    dnums = lax.GatherDimensionNumbers(
        offset_dims=(), collapsed_slice_dims=(0,), start_index_map=(0,))
    return lax.gather(v, jnp.full((16, 1), j, jnp.int32), dnums, (1,),
                      mode=lax.GatherScatterMode.PROMISE_IN_BOUNDS)


_GRP = 16
_NGRP = _QPW // _GRP
_NPAIR = _QPW // 2


def _sc_body(val_hbm, idx_hbm, wgt_hbm, out_hbm, ibuf, wbuf, rb0, rb1, obuf,
             sem_iw, sem_g0, sem_g1, sem_o):
    wid = lax.axis_index("s") * 2 + lax.axis_index("c")
    q0 = wid * _QPW

    def iw_copies(gdst):
        s = (q0 + gdst * _GRP) * 4
        hi = pltpu.make_async_copy(idx_hbm.at[pl.ds(s, _GRP * 4)], ibuf.at[gdst & 1], sem_iw)
        hw = pltpu.make_async_copy(wgt_hbm.at[pl.ds(s, _GRP * 4)], wbuf.at[gdst & 1], sem_iw)
        return hi, hw

    def gather(t, rb, sem):
        gb = (t // _GRP) & 1
        sl = t % _GRP
        return [pltpu.make_async_copy(val_hbm.at[ibuf.at[gb, sl * 4 + c]], rb.at[c], sem)
                for c in range(4)]

    def compute(t, rb):
        gb = (t // _GRP) & 1
        sl = t % _GRP

        def per_head(h, _):
            a0 = []
            a1 = []
            for c in range(4):
                w16 = wbuf[gb, sl * 4 + c, pl.ds(h * 16, 16)]
                acc0 = jnp.zeros((16,), jnp.float32)
                acc1 = jnp.zeros((16,), jnp.float32)
                for j in range(16):
                    wj = _bcast_lane(w16, j)
                    r = rb[c, h * 16 + j]
                    lo = lax.bitcast_convert_type(r << 16, jnp.float32)
                    hi = lax.bitcast_convert_type(r & jnp.int32(-65536), jnp.float32)
                    acc0 = acc0 + wj * lo
                    acc1 = acc1 + wj * hi
                a0.append(acc0)
                a1.append(acc1)
            obuf[gb, sl, pl.ds(h * 32, 16)] = (a0[0] + a0[1]) + (a0[2] + a0[3])
            obuf[gb, sl, pl.ds(h * 32 + 16, 16)] = (a1[0] + a1[1]) + (a1[2] + a1[3])
            return 0

        lax.fori_loop(0, _NH, per_head, 0)

    hi, hw = iw_copies(0)
    hi.start()
    hw.start()

    def pair(i2, _):
        g = i2 // (_GRP // 2)
        slot = i2 % (_GRP // 2)
        a = 2 * i2
        b = a + 1

        @pl.when(slot == 0)
        def _():
            h1, h2 = iw_copies(g)
            h1.wait()
            h2.wait()

            @pl.when(g + 1 < _NGRP)
            def _():
                h3, h4 = iw_copies(g + 1)
                h3.start()
                h4.start()

            @pl.when(g >= 2)
            def _():
                pltpu.make_async_copy(
                    obuf.at[g & 1],
                    out_hbm.at[pl.ds(q0 + (g - 2) * _GRP, _GRP)], sem_o).wait()

            for cp in gather(a, rb0, sem_g0):
                cp.start()

        for cp in gather(b, rb1, sem_g1):
            cp.start()
        for cp in gather(a, rb0, sem_g0):
            cp.wait()
        compute(a, rb0)

        @pl.when(slot < _GRP // 2 - 1)
        def _():
            for cp in gather(a + 2, rb0, sem_g0):
                cp.start()

        for cp in gather(b, rb1, sem_g1):
            cp.wait()
        compute(b, rb1)

        @pl.when(slot == _GRP // 2 - 1)
        def _():
            pltpu.make_async_copy(
                obuf.at[g & 1], out_hbm.at[pl.ds(q0 + g * _GRP, _GRP)], sem_o).start()
        return 0

    lax.fori_loop(0, _NPAIR, pair, 0)
    pltpu.make_async_copy(obuf.at[0], out_hbm.at[pl.ds(q0, _GRP)], sem_o).wait()
    pltpu.make_async_copy(obuf.at[0], out_hbm.at[pl.ds(q0, _GRP)], sem_o).wait()


@functools.cache
def _sc_gather():
    mesh = plsc.VectorSubcoreMesh(core_axis_name="c", subcore_axis_name="s")
    return pl.kernel(
        _sc_body,
        mesh=mesh,
        compiler_params=pltpu.CompilerParams(use_tc_tiling_on_sc=False),
        out_type=jax.ShapeDtypeStruct((_NQ, _D), jnp.float32),
        scratch_types=[
            pltpu.VMEM((2, _GRP * 4, 128), jnp.int32),
            pltpu.VMEM((2, _GRP * 4, 128), jnp.float32),
            pltpu.VMEM((4, 128, _DH // 2), jnp.int32),
            pltpu.VMEM((4, 128, _DH // 2), jnp.int32),
            pltpu.VMEM((2, _GRP, _D), jnp.float32),
            pltpu.SemaphoreType.DMA,
            pltpu.SemaphoreType.DMA,
            pltpu.SemaphoreType.DMA,
            pltpu.SemaphoreType.DMA,
        ],
    )


def _ln_inline(x, g, b):
    m = jnp.mean(x, -1, keepdims=True)
    v = jnp.mean((x - m) ** 2, -1, keepdims=True)
    return (x - m) / jnp.sqrt(v + 1e-5) * g + b


def _kb_body(x_ref, at_ref, wo_ref, bo_ref, g1_ref, be1_ref, w1_ref, bf1_ref,
             w2_ref, bf2_ref, g2_ref, be2_ref, o_ref):
    x = x_ref[0]
    a = at_ref[0]
    h1 = x + jnp.dot(a, wo_ref[...], preferred_element_type=jnp.float32) + bo_ref[...]
    x1 = _ln_inline(h1, g1_ref[...], be1_ref[...])
    ff = jnp.maximum(jnp.dot(x1, w1_ref[...], preferred_element_type=jnp.float32) + bf1_ref[...], 0.0)
    ff2 = jnp.dot(ff, w2_ref[...], preferred_element_type=jnp.float32) + bf2_ref[...]
    o_ref[0] = _ln_inline(x1 + ff2, g2_ref[...], be2_ref[...])


def _kernel_b(x, attn, wo, bo, g1, be1, w1, bf1, w2, bf2, g2, be2, interpret=False):
    c1 = lambda i, j: (0, 0)
    return pl.pallas_call(
        _kb_body,
        grid=(_B, _NBLK),
        in_specs=[
            pl.BlockSpec((1, _CHUNK, _D), lambda i, j: (i, j, 0)),
            pl.BlockSpec((1, _CHUNK, _D), lambda i, j: (i, j, 0)),
            pl.BlockSpec((_D, _D), c1),
            pl.BlockSpec((1, _D), c1),
            pl.BlockSpec((1, _D), c1),
            pl.BlockSpec((1, _D), c1),
            pl.BlockSpec((_D, 1024), c1),
            pl.BlockSpec((1, 1024), c1),
            pl.BlockSpec((1024, _D), c1),
            pl.BlockSpec((1, _D), c1),
            pl.BlockSpec((1, _D), c1),
            pl.BlockSpec((1, _D), c1),
        ],
        out_specs=pl.BlockSpec((1, _CHUNK, _D), lambda i, j: (i, j, 0)),
        out_shape=jax.ShapeDtypeStruct((_B, _LQ, _D), jnp.float32),
        interpret=interpret,
    )(x, attn, wo, bo, g1, be1, w1, bf1, w2, bf2, g2, be2)


def kernel(src0, src1, src2, src3, pos0, pos1, pos2, pos3, level_embed,
           Wso, bso, Waw, baw, Wv, bv, Wo, bo, g1, be1, W1, bf1, W2, bf2, g2, be2):
    srcs = [src0, src1, src2, src3]
    poss = [pos0, pos1, pos2, pos3]
    x = jnp.concatenate([s.reshape(_B, _D, -1).transpose(0, 2, 1) for s in srcs], 1)
    qpos = jnp.concatenate(
        [p.reshape(_B, _D, -1).transpose(0, 2, 1) + level_embed[i][None, None, :]
         for i, p in enumerate(poss)], 1)

    perm = jnp.asarray(_PERM)
    permv = jnp.asarray(_PERM_V)
    refx = jnp.asarray(_REFX)
    refy = jnp.asarray(_REFY)
    sc = _sc_gather()

    for li in range(_NLAYERS):
        wso_p = Wso[li][:, perm]
        bso_p = bso[li][perm][None]
        value, idx, wgt = _kernel_a(x, qpos, refx, refy, wso_p, bso_p,
                                    Waw[li], baw[li][None],
                                    Wv[li][:, permv], bv[li][permv][None])
        val_i32 = lax.bitcast_convert_type(
            value.reshape(_NQ * _NH, _DH // 2, 2), jnp.int32)
        attn = sc(val_i32,
                  idx.reshape(_NQ * 4, 128),
                  wgt.reshape(_NQ * 4, 128))
        x = _kernel_b(x, attn.reshape(_B, _LQ, _D), Wo[li], bo[li][None],
                      g1[li][None], be1[li][None], W1[li], bf1[li][None],
                      W2[li], bf2[li][None], g2[li][None], be2[li][None])
    return x


# speedup vs baseline: 5.3695x; 5.3695x over previous
import functools

import jax
import jax.numpy as jnp
import numpy as np
from jax import lax
from jax.experimental import pallas as pl
from jax.experimental.pallas import tpu as pltpu
from jax.experimental.pallas import tpu_sc as plsc

_SHAPES = [(128, 128), (64, 64), (32, 32), (16, 16)]
_B = 2
_D = 256
_NLAYERS = 3
_NH = 8
_NL = 4
_NP = 4
_DH = _D // _NH
_LQ = sum(h * w for h, w in _SHAPES)
_CHUNK = 1280
_NBLK = _LQ // _CHUNK
_NQ = _B * _LQ
_NWORK = 32
_QPW = _NQ // _NWORK

_lane = np.arange(128)
_h_of = _lane // 16
_l_of = (_lane // 4) % 4
_W_LVL = np.array([w for h, w in _SHAPES], np.float32)
_H_LVL = np.array([h for h, w in _SHAPES], np.float32)
_BASE_LVL = np.cumsum([0] + [h * w for h, w in _SHAPES])[:4]

_SW = _W_LVL[_l_of].reshape(1, 128)
_SH = _H_LVL[_l_of].reshape(1, 128)
_BASE = _BASE_LVL[_l_of].astype(np.int32).reshape(1, 128)
_HL = _h_of.astype(np.int32).reshape(1, 128)

_G = np.kron(np.eye(8, dtype=np.float32), np.ones((16, 16), np.float32))

_PERM = np.empty(256, np.int32)
for _c in range(2):
    for _hh in range(8):
        for _ll in range(4):
            for _pp in range(4):
                _PERM[_c * 128 + _hh * 16 + _ll * 4 + _pp] = ((_hh * 4 + _ll) * 4 + _pp) * 2 + _c

_PERM_V = np.empty(256, np.int32)
for _hh in range(8):
    for _k in range(16):
        _PERM_V[_hh * 16 + _k] = _hh * 32 + _k
        _PERM_V[128 + _hh * 16 + _k] = _hh * 32 + 16 + _k


def _ref_points_np():
    xs, ys = [], []
    for h, w in _SHAPES:
        ry = (np.arange(h, dtype=np.float32) + 0.5) / h
        rx = (np.arange(w, dtype=np.float32) + 0.5) / w
        gy, gx = np.meshgrid(ry, rx, indexing="ij")
        xs.append(gx.reshape(-1))
        ys.append(gy.reshape(-1))
    return np.concatenate(xs), np.concatenate(ys)


_REFX, _REFY = _ref_points_np()
_REFX = _REFX.reshape(_LQ, 1)
_REFY = _REFY.reshape(_LQ, 1)


def _ka_body(x_ref, qp_ref, refx_ref, refy_ref, wso_ref, bso_ref, waw_ref,
             baw_ref, wv_ref, bv_ref, sw_ref, sh_ref, base_ref, hl_ref, g_ref,
             val_ref, idx_ref, wgt_ref):
    b = pl.program_id(0)
    x = x_ref[0]
    q = x + qp_ref[0]
    off = jnp.dot(q, wso_ref[...], preferred_element_type=jnp.float32) + bso_ref[...]
    logits = jnp.dot(q, waw_ref[...], preferred_element_type=jnp.float32) + baw_ref[...]
    m = jnp.max(logits, axis=-1, keepdims=True)
    e = jnp.exp(logits - m)
    gs = lax.dot(e, g_ref[...], precision=lax.Precision.HIGHEST)
    aw = e / gs
    v = jnp.dot(x, wv_ref[...], preferred_element_type=jnp.float32) + bv_ref[...]
    u = lax.bitcast_convert_type(v, jnp.int32)
    r = u + jnp.int32(0x7FFF) + ((u >> 16) & 1)
    lo = (r[:, :128] >> 16) & jnp.int32(0xFFFF)
    hi = r[:, 128:] & jnp.int32(-65536)
    val_ref[0] = lo | hi

    offx = off[:, :128]
    offy = off[:, 128:]
    sw = sw_ref[...]
    sh = sh_ref[...]
    xi = refx_ref[...] * sw + offx - 0.5
    yi = refy_ref[...] * sh + offy - 0.5
    x0 = jnp.floor(xi)
    y0 = jnp.floor(yi)
    fx = xi - x0
    fy = yi - y0
    x0i = x0.astype(jnp.int32)
    y0i = y0.astype(jnp.int32)
    swi = sw.astype(jnp.int32)
    shi = sh.astype(jnp.int32)
    rowbase = b * _LQ + base_ref[...]
    hl = hl_ref[...]

    outs_i = []
    outs_w = []
    for dy in (0, 1):
        wy = (1.0 - fy) if dy == 0 else fy
        yc = y0i + dy
        vy = (yc >= 0) & (yc <= shi - 1)
        cy = jnp.clip(yc, 0, shi - 1)
        for dx in (0, 1):
            wx = (1.0 - fx) if dx == 0 else fx
            xc = x0i + dx
            vx = (xc >= 0) & (xc <= swi - 1)
            cx = jnp.clip(xc, 0, swi - 1)
            wc = wx * wy * (vx & vy).astype(jnp.float32) * aw
            row = (rowbase + cy * swi + cx) * _NH + hl
            outs_i.append(row)
            outs_w.append(wc)
    idx_ref[0] = jnp.concatenate(outs_i, axis=-1)
    wgt_ref[0] = jnp.concatenate(outs_w, axis=-1)


def _kernel_a(x, qpos, refx, refy, wso, bso, waw, baw, wv, bv, interpret=False):
    c1 = lambda i, j: (0, 0)
    return pl.pallas_call(
        _ka_body,
        grid=(_B, _NBLK),
        in_specs=[
            pl.BlockSpec((1, _CHUNK, _D), lambda i, j: (i, j, 0)),
            pl.BlockSpec((1, _CHUNK, _D), lambda i, j: (i, j, 0)),
            pl.BlockSpec((_CHUNK, 1), lambda i, j: (j, 0)),
            pl.BlockSpec((_CHUNK, 1), lambda i, j: (j, 0)),
            pl.BlockSpec((_D, 256), c1),
            pl.BlockSpec((1, 256), c1),
            pl.BlockSpec((_D, 128), c1),
            pl.BlockSpec((1, 128), c1),
            pl.BlockSpec((_D, _D), c1),
            pl.BlockSpec((1, _D), c1),
            pl.BlockSpec((1, 128), c1),
            pl.BlockSpec((1, 128), c1),
            pl.BlockSpec((1, 128), c1),
            pl.BlockSpec((1, 128), c1),
            pl.BlockSpec((128, 128), c1),
        ],
        out_specs=[
            pl.BlockSpec((1, _CHUNK, 128), lambda i, j: (i, j, 0)),
            pl.BlockSpec((1, _CHUNK, 512), lambda i, j: (i, j, 0)),
            pl.BlockSpec((1, _CHUNK, 512), lambda i, j: (i, j, 0)),
        ],
        out_shape=[
            jax.ShapeDtypeStruct((_B, _LQ, 128), jnp.int32),
            jax.ShapeDtypeStruct((_B, _LQ, 512), jnp.int32),
            jax.ShapeDtypeStruct((_B, _LQ, 512), jnp.float32),
        ],
        interpret=interpret,
    )(x, qpos, refx, refy, wso, bso, waw, baw, wv, bv,
      jnp.asarray(_SW), jnp.asarray(_SH), jnp.asarray(_BASE), jnp.asarray(_HL),
      jnp.asarray(_G))


def _bcast_lane(v, j):
    dnums = lax.GatherDimensionNumbers(
        offset_dims=(), collapsed_slice_dims=(0,), start_index_map=(0,))
    return lax.gather(v, jnp.full((16, 1), j, jnp.int32), dnums, (1,),
                      mode=lax.GatherScatterMode.PROMISE_IN_BOUNDS)


_GRP = 16
_NGRP = _QPW // _GRP
_NPAIR = _QPW // 2


def _sc_body(val_hbm, idx_hbm, wgt_hbm, out_hbm, ibuf, wbuf, rb0, rb1, obuf,
             sem_iw, sem_g0, sem_g1, sem_o):
    wid = lax.axis_index("s") * 2 + lax.axis_index("c")
    q0 = wid * _QPW

    def iw_copies(gdst):
        s = (q0 + gdst * _GRP) * 4
        hi = pltpu.make_async_copy(idx_hbm.at[pl.ds(s, _GRP * 4)], ibuf.at[gdst & 1], sem_iw)
        hw = pltpu.make_async_copy(wgt_hbm.at[pl.ds(s, _GRP * 4)], wbuf.at[gdst & 1], sem_iw)
        return hi, hw

    def gather(t, rb, sem):
        gb = (t // _GRP) & 1
        sl = t % _GRP
        return [pltpu.make_async_copy(val_hbm.at[ibuf.at[gb, sl * 4 + c]], rb.at[c], sem)
                for c in range(4)]

    def compute(t, rb):
        gb = (t // _GRP) & 1
        sl = t % _GRP

        def per_head(h, _):
            a0 = []
            a1 = []
            for c in range(4):
                w16 = wbuf[gb, sl * 4 + c, pl.ds(h * 16, 16)]
                acc0 = jnp.zeros((16,), jnp.float32)
                acc1 = jnp.zeros((16,), jnp.float32)
                for j in range(16):
                    wj = _bcast_lane(w16, j)
                    r = rb[c, h * 16 + j]
                    lo = lax.bitcast_convert_type(r << 16, jnp.float32)
                    hi = lax.bitcast_convert_type(r & jnp.int32(-65536), jnp.float32)
                    acc0 = acc0 + wj * lo
                    acc1 = acc1 + wj * hi
                a0.append(acc0)
                a1.append(acc1)
            obuf[gb, sl, pl.ds(h * 32, 16)] = (a0[0] + a0[1]) + (a0[2] + a0[3])
            obuf[gb, sl, pl.ds(h * 32 + 16, 16)] = (a1[0] + a1[1]) + (a1[2] + a1[3])
            return 0

        lax.fori_loop(0, _NH, per_head, 0)

    hi, hw = iw_copies(0)
    hi.start()
    hw.start()

    def pair(i2, _):
        g = i2 // (_GRP // 2)
        slot = i2 % (_GRP // 2)
        a = 2 * i2
        b = a + 1

        @pl.when(slot == 0)
        def _():
            h1, h2 = iw_copies(g)
            h1.wait()
            h2.wait()

            @pl.when(g + 1 < _NGRP)
            def _():
                h3, h4 = iw_copies(g + 1)
                h3.start()
                h4.start()

            @pl.when(g >= 2)
            def _():
                pltpu.make_async_copy(
                    obuf.at[g & 1],
                    out_hbm.at[pl.ds(q0 + (g - 2) * _GRP, _GRP)], sem_o).wait()

            for cp in gather(a, rb0, sem_g0):
                cp.start()

        for cp in gather(b, rb1, sem_g1):
            cp.start()
        for cp in gather(a, rb0, sem_g0):
            cp.wait()
        compute(a, rb0)

        @pl.when(slot < _GRP // 2 - 1)
        def _():
            for cp in gather(a + 2, rb0, sem_g0):
                cp.start()

        for cp in gather(b, rb1, sem_g1):
            cp.wait()
        compute(b, rb1)

        @pl.when(slot == _GRP // 2 - 1)
        def _():
            pltpu.make_async_copy(
                obuf.at[g & 1], out_hbm.at[pl.ds(q0 + g * _GRP, _GRP)], sem_o).start()
        return 0

    lax.fori_loop(0, _NPAIR, pair, 0)
    pltpu.make_async_copy(obuf.at[0], out_hbm.at[pl.ds(q0, _GRP)], sem_o).wait()
    pltpu.make_async_copy(obuf.at[0], out_hbm.at[pl.ds(q0, _GRP)], sem_o).wait()


@functools.cache
def _sc_gather():
    mesh = plsc.VectorSubcoreMesh(core_axis_name="c", subcore_axis_name="s")
    return pl.kernel(
        _sc_body,
        mesh=mesh,
        compiler_params=pltpu.CompilerParams(use_tc_tiling_on_sc=False),
        out_type=jax.ShapeDtypeStruct((_NQ, _D), jnp.float32),
        scratch_types=[
            pltpu.VMEM((2, _GRP * 4, 128), jnp.int32),
            pltpu.VMEM((2, _GRP * 4, 128), jnp.float32),
            pltpu.VMEM((4, 128, _DH // 2), jnp.int32),
            pltpu.VMEM((4, 128, _DH // 2), jnp.int32),
            pltpu.VMEM((2, _GRP, _D), jnp.float32),
            pltpu.SemaphoreType.DMA,
            pltpu.SemaphoreType.DMA,
            pltpu.SemaphoreType.DMA,
            pltpu.SemaphoreType.DMA,
        ],
    )


def _ln_inline(x, g, b):
    m = jnp.mean(x, -1, keepdims=True)
    v = jnp.mean((x - m) ** 2, -1, keepdims=True)
    return (x - m) / jnp.sqrt(v + 1e-5) * g + b


def _kb_body(x_ref, at_ref, wo_ref, bo_ref, g1_ref, be1_ref, w1_ref, bf1_ref,
             w2_ref, bf2_ref, g2_ref, be2_ref, o_ref):
    x = x_ref[0]
    a = at_ref[0]
    h1 = x + jnp.dot(a, wo_ref[...], preferred_element_type=jnp.float32) + bo_ref[...]
    x1 = _ln_inline(h1, g1_ref[...], be1_ref[...])
    ff = jnp.maximum(jnp.dot(x1, w1_ref[...], preferred_element_type=jnp.float32) + bf1_ref[...], 0.0)
    ff2 = jnp.dot(ff, w2_ref[...], preferred_element_type=jnp.float32) + bf2_ref[...]
    o_ref[0] = _ln_inline(x1 + ff2, g2_ref[...], be2_ref[...])


def _kernel_b(x, attn, wo, bo, g1, be1, w1, bf1, w2, bf2, g2, be2, interpret=False):
    c1 = lambda i, j: (0, 0)
    return pl.pallas_call(
        _kb_body,
        grid=(_B, _NBLK),
        in_specs=[
            pl.BlockSpec((1, _CHUNK, _D), lambda i, j: (i, j, 0)),
            pl.BlockSpec((1, _CHUNK, _D), lambda i, j: (i, j, 0)),
            pl.BlockSpec((_D, _D), c1),
            pl.BlockSpec((1, _D), c1),
            pl.BlockSpec((1, _D), c1),
            pl.BlockSpec((1, _D), c1),
            pl.BlockSpec((_D, 1024), c1),
            pl.BlockSpec((1, 1024), c1),
            pl.BlockSpec((1024, _D), c1),
            pl.BlockSpec((1, _D), c1),
            pl.BlockSpec((1, _D), c1),
            pl.BlockSpec((1, _D), c1),
        ],
        out_specs=pl.BlockSpec((1, _CHUNK, _D), lambda i, j: (i, j, 0)),
        out_shape=jax.ShapeDtypeStruct((_B, _LQ, _D), jnp.float32),
        interpret=interpret,
    )(x, attn, wo, bo, g1, be1, w1, bf1, w2, bf2, g2, be2)


def kernel(src0, src1, src2, src3, pos0, pos1, pos2, pos3, level_embed,
           Wso, bso, Waw, baw, Wv, bv, Wo, bo, g1, be1, W1, bf1, W2, bf2, g2, be2):
    srcs = [src0, src1, src2, src3]
    poss = [pos0, pos1, pos2, pos3]
    x = jnp.concatenate([s.reshape(_B, _D, -1).transpose(0, 2, 1) for s in srcs], 1)
    qpos = jnp.concatenate(
        [p.reshape(_B, _D, -1).transpose(0, 2, 1) + level_embed[i][None, None, :]
         for i, p in enumerate(poss)], 1)

    perm = jnp.asarray(_PERM)
    permv = jnp.asarray(_PERM_V)
    refx = jnp.asarray(_REFX)
    refy = jnp.asarray(_REFY)
    sc = _sc_gather()

    for li in range(_NLAYERS):
        wso_p = Wso[li][:, perm]
        bso_p = bso[li][perm][None]
        value, idx, wgt = _kernel_a(x, qpos, refx, refy, wso_p, bso_p,
                                    Waw[li], baw[li][None],
                                    Wv[li][:, permv], bv[li][permv][None])
        attn = sc(value.reshape(_NQ * _NH, _DH // 2),
                  idx.reshape(_NQ * 4, 128),
                  wgt.reshape(_NQ * 4, 128))
        x = _kernel_b(x, attn.reshape(_B, _LQ, _D), Wo[li], bo[li][None],
                      g1[li][None], be1[li][None], W1[li], bf1[li][None],
                      W2[li], bf2[li][None], g2[li][None], be2[li][None])
    return x


# speedup vs baseline: 5.5179x; 1.0276x over previous
import functools

import jax
import jax.numpy as jnp
import numpy as np
from jax import lax
from jax.experimental import pallas as pl
from jax.experimental.pallas import tpu as pltpu
from jax.experimental.pallas import tpu_sc as plsc

_SHAPES = [(128, 128), (64, 64), (32, 32), (16, 16)]
_B = 2
_D = 256
_NLAYERS = 3
_NH = 8
_NL = 4
_NP = 4
_DH = _D // _NH
_LQ = sum(h * w for h, w in _SHAPES)
_CHUNK = 1280
_NBLK = _LQ // _CHUNK
_NQ = _B * _LQ
_NWORK = 32
_QPW = _NQ // _NWORK

_lane = np.arange(128)
_h_of = _lane // 16
_l_of = (_lane // 4) % 4
_W_LVL = np.array([w for h, w in _SHAPES], np.float32)
_H_LVL = np.array([h for h, w in _SHAPES], np.float32)
_BASE_LVL = np.cumsum([0] + [h * w for h, w in _SHAPES])[:4]

_SW = _W_LVL[_l_of].reshape(1, 128)
_SH = _H_LVL[_l_of].reshape(1, 128)
_BASE = _BASE_LVL[_l_of].astype(np.int32).reshape(1, 128)
_HL = _h_of.astype(np.int32).reshape(1, 128)

_G = np.kron(np.eye(8, dtype=np.float32), np.ones((16, 16), np.float32))

_PERM = np.empty(256, np.int32)
for _c in range(2):
    for _hh in range(8):
        for _ll in range(4):
            for _pp in range(4):
                _PERM[_c * 128 + _hh * 16 + _ll * 4 + _pp] = ((_hh * 4 + _ll) * 4 + _pp) * 2 + _c

_PERM_V = np.empty(256, np.int32)
for _hh in range(8):
    for _k in range(16):
        _PERM_V[_hh * 16 + _k] = _hh * 32 + _k
        _PERM_V[128 + _hh * 16 + _k] = _hh * 32 + 16 + _k


def _ref_points_np():
    xs, ys = [], []
    for h, w in _SHAPES:
        ry = (np.arange(h, dtype=np.float32) + 0.5) / h
        rx = (np.arange(w, dtype=np.float32) + 0.5) / w
        gy, gx = np.meshgrid(ry, rx, indexing="ij")
        xs.append(gx.reshape(-1))
        ys.append(gy.reshape(-1))
    return np.concatenate(xs), np.concatenate(ys)


_REFX, _REFY = _ref_points_np()
_REFX = _REFX.reshape(_LQ, 1)
_REFY = _REFY.reshape(_LQ, 1)


def _ka_body(x_ref, qp_ref, refx_ref, refy_ref, wso_ref, bso_ref, waw_ref,
             baw_ref, wv_ref, bv_ref, sw_ref, sh_ref, base_ref, hl_ref, g_ref,
             val_ref, idx_ref, wgt_ref):
    b = pl.program_id(0)
    x = x_ref[0]
    q = x + qp_ref[0]
    off = jnp.dot(q, wso_ref[...], preferred_element_type=jnp.float32) + bso_ref[...]
    logits = jnp.dot(q, waw_ref[...], preferred_element_type=jnp.float32) + baw_ref[...]
    m = jnp.max(logits, axis=-1, keepdims=True)
    e = jnp.exp(logits - m)
    gs = lax.dot(e, g_ref[...], precision=lax.Precision.HIGHEST)
    aw = e / gs
    v = jnp.dot(x, wv_ref[...], preferred_element_type=jnp.float32) + bv_ref[...]
    u = lax.bitcast_convert_type(v, jnp.int32)
    r = u + jnp.int32(0x7FFF) + ((u >> 16) & 1)
    lo = (r[:, :128] >> 16) & jnp.int32(0xFFFF)
    hi = r[:, 128:] & jnp.int32(-65536)
    val_ref[0] = lo | hi

    offx = off[:, :128]
    offy = off[:, 128:]
    sw = sw_ref[...]
    sh = sh_ref[...]
    xi = refx_ref[...] * sw + offx - 0.5
    yi = refy_ref[...] * sh + offy - 0.5
    x0 = jnp.floor(xi)
    y0 = jnp.floor(yi)
    fx = xi - x0
    fy = yi - y0
    x0i = x0.astype(jnp.int32)
    y0i = y0.astype(jnp.int32)
    swi = sw.astype(jnp.int32)
    shi = sh.astype(jnp.int32)
    rowbase = b * _LQ + base_ref[...]
    hl = hl_ref[...]

    outs_i = []
    outs_w = []
    for dy in (0, 1):
        wy = (1.0 - fy) if dy == 0 else fy
        yc = y0i + dy
        vy = (yc >= 0) & (yc <= shi - 1)
        cy = jnp.clip(yc, 0, shi - 1)
        for dx in (0, 1):
            wx = (1.0 - fx) if dx == 0 else fx
            xc = x0i + dx
            vx = (xc >= 0) & (xc <= swi - 1)
            cx = jnp.clip(xc, 0, swi - 1)
            wc = wx * wy * (vx & vy).astype(jnp.float32) * aw
            row = (rowbase + cy * swi + cx) * _NH + hl
            outs_i.append(row)
            outs_w.append(wc)
    idx_ref[0] = jnp.concatenate(outs_i, axis=-1)
    wgt_ref[0] = jnp.concatenate(outs_w, axis=-1)


def _kernel_a(x, qpos, refx, refy, wso, bso, waw, baw, wv, bv, interpret=False):
    c1 = lambda i, j: (0, 0)
    return pl.pallas_call(
        _ka_body,
        grid=(_B, _NBLK),
        in_specs=[
            pl.BlockSpec((1, _CHUNK, _D), lambda i, j: (i, j, 0)),
            pl.BlockSpec((1, _CHUNK, _D), lambda i, j: (i, j, 0)),
            pl.BlockSpec((_CHUNK, 1), lambda i, j: (j, 0)),
            pl.BlockSpec((_CHUNK, 1), lambda i, j: (j, 0)),
            pl.BlockSpec((_D, 256), c1),
            pl.BlockSpec((1, 256), c1),
            pl.BlockSpec((_D, 128), c1),
            pl.BlockSpec((1, 128), c1),
            pl.BlockSpec((_D, _D), c1),
            pl.BlockSpec((1, _D), c1),
            pl.BlockSpec((1, 128), c1),
            pl.BlockSpec((1, 128), c1),
            pl.BlockSpec((1, 128), c1),
            pl.BlockSpec((1, 128), c1),
            pl.BlockSpec((128, 128), c1),
        ],
        out_specs=[
            pl.BlockSpec((1, _CHUNK, 128), lambda i, j: (i, j, 0)),
            pl.BlockSpec((1, _CHUNK, 512), lambda i, j: (i, j, 0)),
            pl.BlockSpec((1, _CHUNK, 512), lambda i, j: (i, j, 0)),
        ],
        out_shape=[
            jax.ShapeDtypeStruct((_B, _LQ, 128), jnp.int32),
            jax.ShapeDtypeStruct((_B, _LQ, 512), jnp.int32),
            jax.ShapeDtypeStruct((_B, _LQ, 512), jnp.float32),
        ],
        interpret=interpret,
    )(x, qpos, refx, refy, wso, bso, waw, baw, wv, bv,
      jnp.asarray(_SW), jnp.asarray(_SH), jnp.asarray(_BASE), jnp.asarray(_HL),
      jnp.asarray(_G))


def _bcast_lane(v, j):
    dnums = lax.GatherDimensionNumbers(
        offset_dims=(), collapsed_slice_dims=(0,), start_index_map=(0,))
    return lax.gather(v, jnp.full((16, 1), j, jnp.int32), dnums, (1,),
                      mode=lax.GatherScatterMode.PROMISE_IN_BOUNDS)


_GRP = 16
_NGRP = _QPW // _GRP
_NPAIR = _QPW // 2


def _sc_body(val_hbm, idx_hbm, wgt_hbm, out_hbm, ibuf, wbuf, rb0, rb1, obuf,
             sem_iw, sem_g0, sem_g1, sem_o):
    wid = lax.axis_index("s") * 2 + lax.axis_index("c")
    q0 = wid * _QPW

    def iw_copies(gdst):
        s = (q0 + gdst * _GRP) * 4
        hi = pltpu.make_async_copy(idx_hbm.at[pl.ds(s, _GRP * 4)], ibuf.at[gdst & 1], sem_iw)
        hw = pltpu.make_async_copy(wgt_hbm.at[pl.ds(s, _GRP * 4)], wbuf.at[gdst & 1], sem_iw)
        return hi, hw

    def gather(t, rb, sem):
        gb = (t // _GRP) & 1
        sl = t % _GRP
        return [pltpu.make_async_copy(val_hbm.at[ibuf.at[gb, sl * 4 + c]], rb.at[c], sem)
                for c in range(4)]

    def compute(t, rb):
        gb = (t // _GRP) & 1
        sl = t % _GRP

        def per_head(h, _):
            a0 = []
            a1 = []
            for c in range(4):
                w16 = wbuf[gb, sl * 4 + c, pl.ds(h * 16, 16)]
                acc0 = jnp.zeros((16,), jnp.float32)
                acc1 = jnp.zeros((16,), jnp.float32)
                for j in range(16):
                    wj = _bcast_lane(w16, j)
                    r = rb[c, h * 16 + j]
                    lo = lax.bitcast_convert_type(r << 16, jnp.float32)
                    hi = lax.bitcast_convert_type(r, jnp.float32)
                    acc0 = acc0 + wj * lo
                    acc1 = acc1 + wj * hi
                a0.append(acc0)
                a1.append(acc1)
            obuf[gb, sl, pl.ds(h * 32, 16)] = (a0[0] + a0[1]) + (a0[2] + a0[3])
            obuf[gb, sl, pl.ds(h * 32 + 16, 16)] = (a1[0] + a1[1]) + (a1[2] + a1[3])
            return 0

        lax.fori_loop(0, _NH, per_head, 0)

    hi, hw = iw_copies(0)
    hi.start()
    hw.start()

    def pair(i2, _):
        g = i2 // (_GRP // 2)
        slot = i2 % (_GRP // 2)
        a = 2 * i2
        b = a + 1

        @pl.when(slot == 0)
        def _():
            h1, h2 = iw_copies(g)
            h1.wait()
            h2.wait()

            @pl.when(g + 1 < _NGRP)
            def _():
                h3, h4 = iw_copies(g + 1)
                h3.start()
                h4.start()

            @pl.when(g >= 2)
            def _():
                pltpu.make_async_copy(
                    obuf.at[g & 1],
                    out_hbm.at[pl.ds(q0 + (g - 2) * _GRP, _GRP)], sem_o).wait()

            for cp in gather(a, rb0, sem_g0):
                cp.start()

        for cp in gather(b, rb1, sem_g1):
            cp.start()
        for cp in gather(a, rb0, sem_g0):
            cp.wait()
        compute(a, rb0)

        @pl.when(slot < _GRP // 2 - 1)
        def _():
            for cp in gather(a + 2, rb0, sem_g0):
                cp.start()

        for cp in gather(b, rb1, sem_g1):
            cp.wait()
        compute(b, rb1)

        @pl.when(slot == _GRP // 2 - 1)
        def _():
            pltpu.make_async_copy(
                obuf.at[g & 1], out_hbm.at[pl.ds(q0 + g * _GRP, _GRP)], sem_o).start()
        return 0

    lax.fori_loop(0, _NPAIR, pair, 0)
    pltpu.make_async_copy(obuf.at[0], out_hbm.at[pl.ds(q0, _GRP)], sem_o).wait()
    pltpu.make_async_copy(obuf.at[0], out_hbm.at[pl.ds(q0, _GRP)], sem_o).wait()


@functools.cache
def _sc_gather():
    mesh = plsc.VectorSubcoreMesh(core_axis_name="c", subcore_axis_name="s")
    return pl.kernel(
        _sc_body,
        mesh=mesh,
        compiler_params=pltpu.CompilerParams(use_tc_tiling_on_sc=False),
        out_type=jax.ShapeDtypeStruct((_NQ, _D), jnp.float32),
        scratch_types=[
            pltpu.VMEM((2, _GRP * 4, 128), jnp.int32),
            pltpu.VMEM((2, _GRP * 4, 128), jnp.float32),
            pltpu.VMEM((4, 128, _DH // 2), jnp.int32),
            pltpu.VMEM((4, 128, _DH // 2), jnp.int32),
            pltpu.VMEM((2, _GRP, _D), jnp.float32),
            pltpu.SemaphoreType.DMA,
            pltpu.SemaphoreType.DMA,
            pltpu.SemaphoreType.DMA,
            pltpu.SemaphoreType.DMA,
        ],
    )


def _ln_inline(x, g, b):
    m = jnp.mean(x, -1, keepdims=True)
    v = jnp.mean((x - m) ** 2, -1, keepdims=True)
    return (x - m) / jnp.sqrt(v + 1e-5) * g + b


def _kb_body(x_ref, at_ref, wo_ref, bo_ref, g1_ref, be1_ref, w1_ref, bf1_ref,
             w2_ref, bf2_ref, g2_ref, be2_ref, o_ref):
    x = x_ref[0]
    a = at_ref[0]
    h1 = x + jnp.dot(a, wo_ref[...], preferred_element_type=jnp.float32) + bo_ref[...]
    x1 = _ln_inline(h1, g1_ref[...], be1_ref[...])
    ff = jnp.maximum(jnp.dot(x1, w1_ref[...], preferred_element_type=jnp.float32) + bf1_ref[...], 0.0)
    ff2 = jnp.dot(ff, w2_ref[...], preferred_element_type=jnp.float32) + bf2_ref[...]
    o_ref[0] = _ln_inline(x1 + ff2, g2_ref[...], be2_ref[...])


def _kernel_b(x, attn, wo, bo, g1, be1, w1, bf1, w2, bf2, g2, be2, interpret=False):
    c1 = lambda i, j: (0, 0)
    return pl.pallas_call(
        _kb_body,
        grid=(_B, _NBLK),
        in_specs=[
            pl.BlockSpec((1, _CHUNK, _D), lambda i, j: (i, j, 0)),
            pl.BlockSpec((1, _CHUNK, _D), lambda i, j: (i, j, 0)),
            pl.BlockSpec((_D, _D), c1),
            pl.BlockSpec((1, _D), c1),
            pl.BlockSpec((1, _D), c1),
            pl.BlockSpec((1, _D), c1),
            pl.BlockSpec((_D, 1024), c1),
            pl.BlockSpec((1, 1024), c1),
            pl.BlockSpec((1024, _D), c1),
            pl.BlockSpec((1, _D), c1),
            pl.BlockSpec((1, _D), c1),
            pl.BlockSpec((1, _D), c1),
        ],
        out_specs=pl.BlockSpec((1, _CHUNK, _D), lambda i, j: (i, j, 0)),
        out_shape=jax.ShapeDtypeStruct((_B, _LQ, _D), jnp.float32),
        interpret=interpret,
    )(x, attn, wo, bo, g1, be1, w1, bf1, w2, bf2, g2, be2)


def kernel(src0, src1, src2, src3, pos0, pos1, pos2, pos3, level_embed,
           Wso, bso, Waw, baw, Wv, bv, Wo, bo, g1, be1, W1, bf1, W2, bf2, g2, be2):
    srcs = [src0, src1, src2, src3]
    poss = [pos0, pos1, pos2, pos3]
    x = jnp.concatenate([s.reshape(_B, _D, -1).transpose(0, 2, 1) for s in srcs], 1)
    qpos = jnp.concatenate(
        [p.reshape(_B, _D, -1).transpose(0, 2, 1) + level_embed[i][None, None, :]
         for i, p in enumerate(poss)], 1)

    perm = jnp.asarray(_PERM)
    permv = jnp.asarray(_PERM_V)
    refx = jnp.asarray(_REFX)
    refy = jnp.asarray(_REFY)
    sc = _sc_gather()

    for li in range(_NLAYERS):
        wso_p = Wso[li][:, perm]
        bso_p = bso[li][perm][None]
        value, idx, wgt = _kernel_a(x, qpos, refx, refy, wso_p, bso_p,
                                    Waw[li], baw[li][None],
                                    Wv[li][:, permv], bv[li][permv][None])
        attn = sc(value.reshape(_NQ * _NH, _DH // 2),
                  idx.reshape(_NQ * 4, 128),
                  wgt.reshape(_NQ * 4, 128))
        x = _kernel_b(x, attn.reshape(_B, _LQ, _D), Wo[li], bo[li][None],
                      g1[li][None], be1[li][None], W1[li], bf1[li][None],
                      W2[li], bf2[li][None], g2[li][None], be2[li][None])
    return x


# speedup vs baseline: 5.8529x; 1.0607x over previous
import functools

import jax
import jax.numpy as jnp
import numpy as np
from jax import lax
from jax.experimental import pallas as pl
from jax.experimental.pallas import tpu as pltpu
from jax.experimental.pallas import tpu_sc as plsc

_SHAPES = [(128, 128), (64, 64), (32, 32), (16, 16)]
_B = 2
_D = 256
_NLAYERS = 3
_NH = 8
_NL = 4
_NP = 4
_DH = _D // _NH
_LQ = sum(h * w for h, w in _SHAPES)
_CHUNK = 1280
_NBLK = _LQ // _CHUNK
_NQ = _B * _LQ
_NWORK = 32
_QPW = _NQ // _NWORK

_lane = np.arange(128)
_h_of = _lane // 16
_l_of = (_lane // 4) % 4
_W_LVL = np.array([w for h, w in _SHAPES], np.float32)
_H_LVL = np.array([h for h, w in _SHAPES], np.float32)
_BASE_LVL = np.cumsum([0] + [h * w for h, w in _SHAPES])[:4]

_SW = _W_LVL[_l_of].reshape(1, 128)
_SH = _H_LVL[_l_of].reshape(1, 128)
_BASE = _BASE_LVL[_l_of].astype(np.int32).reshape(1, 128)
_HL = _h_of.astype(np.int32).reshape(1, 128)

_G = np.kron(np.eye(8, dtype=np.float32), np.ones((16, 16), np.float32))

_PERM = np.empty(256, np.int32)
for _c in range(2):
    for _hh in range(8):
        for _ll in range(4):
            for _pp in range(4):
                _PERM[_c * 128 + _hh * 16 + _ll * 4 + _pp] = ((_hh * 4 + _ll) * 4 + _pp) * 2 + _c

_PERM_V = np.empty(256, np.int32)
for _hh in range(8):
    for _k in range(16):
        _PERM_V[_hh * 16 + _k] = _hh * 32 + _k
        _PERM_V[128 + _hh * 16 + _k] = _hh * 32 + 16 + _k


def _ref_points_np():
    xs, ys = [], []
    for h, w in _SHAPES:
        ry = (np.arange(h, dtype=np.float32) + 0.5) / h
        rx = (np.arange(w, dtype=np.float32) + 0.5) / w
        gy, gx = np.meshgrid(ry, rx, indexing="ij")
        xs.append(gx.reshape(-1))
        ys.append(gy.reshape(-1))
    return np.concatenate(xs), np.concatenate(ys)


_REFX, _REFY = _ref_points_np()
_REFX = _REFX.reshape(_LQ, 1)
_REFY = _REFY.reshape(_LQ, 1)


def _ka_body(x_ref, qp_ref, refx_ref, refy_ref, wso_ref, bso_ref, waw_ref,
             baw_ref, wv_ref, bv_ref, sw_ref, sh_ref, base_ref, hl_ref, g_ref,
             val_ref, idx_ref, wgt_ref):
    b = pl.program_id(0)
    x = x_ref[0]
    q = x + qp_ref[0]
    off = jnp.dot(q, wso_ref[...], preferred_element_type=jnp.float32) + bso_ref[...]
    logits = jnp.dot(q, waw_ref[...], preferred_element_type=jnp.float32) + baw_ref[...]
    m = jnp.max(logits, axis=-1, keepdims=True)
    e = jnp.exp(logits - m)
    gs = lax.dot(e, g_ref[...], precision=lax.Precision.HIGHEST)
    aw = e / gs
    v = jnp.dot(x, wv_ref[...], preferred_element_type=jnp.float32) + bv_ref[...]
    u = lax.bitcast_convert_type(v, jnp.int32)
    r = u + jnp.int32(0x7FFF) + ((u >> 16) & 1)
    lo = (r[:, :128] >> 16) & jnp.int32(0xFFFF)
    hi = r[:, 128:] & jnp.int32(-65536)
    val_ref[0] = lo | hi

    offx = off[:, :128]
    offy = off[:, 128:]
    sw = sw_ref[...]
    sh = sh_ref[...]
    xi = refx_ref[...] * sw + offx - 0.5
    yi = refy_ref[...] * sh + offy - 0.5
    x0 = jnp.floor(xi)
    y0 = jnp.floor(yi)
    fx = xi - x0
    fy = yi - y0
    x0i = x0.astype(jnp.int32)
    y0i = y0.astype(jnp.int32)
    swi = sw.astype(jnp.int32)
    shi = sh.astype(jnp.int32)
    rowbase = b * _LQ + base_ref[...]
    hl = hl_ref[...]

    outs_i = []
    outs_w = []
    for dy in (0, 1):
        wy = (1.0 - fy) if dy == 0 else fy
        yc = y0i + dy
        vy = (yc >= 0) & (yc <= shi - 1)
        cy = jnp.clip(yc, 0, shi - 1)
        for dx in (0, 1):
            wx = (1.0 - fx) if dx == 0 else fx
            xc = x0i + dx
            vx = (xc >= 0) & (xc <= swi - 1)
            cx = jnp.clip(xc, 0, swi - 1)
            wc = wx * wy * (vx & vy).astype(jnp.float32) * aw
            row = (rowbase + cy * swi + cx) * _NH + hl
            outs_i.append(row)
            outs_w.append(wc)
    idx_ref[0] = jnp.concatenate(outs_i, axis=-1)
    wgt_ref[0] = jnp.concatenate(outs_w, axis=-1)


def _kernel_a(x, qpos, refx, refy, wso, bso, waw, baw, wv, bv, interpret=False):
    c1 = lambda i, j: (0, 0)
    return pl.pallas_call(
        _ka_body,
        grid=(_B, _NBLK),
        in_specs=[
            pl.BlockSpec((1, _CHUNK, _D), lambda i, j: (i, j, 0)),
            pl.BlockSpec((1, _CHUNK, _D), lambda i, j: (i, j, 0)),
            pl.BlockSpec((_CHUNK, 1), lambda i, j: (j, 0)),
            pl.BlockSpec((_CHUNK, 1), lambda i, j: (j, 0)),
            pl.BlockSpec((_D, 256), c1),
            pl.BlockSpec((1, 256), c1),
            pl.BlockSpec((_D, 128), c1),
            pl.BlockSpec((1, 128), c1),
            pl.BlockSpec((_D, _D), c1),
            pl.BlockSpec((1, _D), c1),
            pl.BlockSpec((1, 128), c1),
            pl.BlockSpec((1, 128), c1),
            pl.BlockSpec((1, 128), c1),
            pl.BlockSpec((1, 128), c1),
            pl.BlockSpec((128, 128), c1),
        ],
        out_specs=[
            pl.BlockSpec((1, _CHUNK, 128), lambda i, j: (i, j, 0)),
            pl.BlockSpec((1, _CHUNK, 512), lambda i, j: (i, j, 0)),
            pl.BlockSpec((1, _CHUNK, 512), lambda i, j: (i, j, 0)),
        ],
        out_shape=[
            jax.ShapeDtypeStruct((_B, _LQ, 128), jnp.int32),
            jax.ShapeDtypeStruct((_B, _LQ, 512), jnp.int32),
            jax.ShapeDtypeStruct((_B, _LQ, 512), jnp.float32),
        ],
        interpret=interpret,
    )(x, qpos, refx, refy, wso, bso, waw, baw, wv, bv,
      jnp.asarray(_SW), jnp.asarray(_SH), jnp.asarray(_BASE), jnp.asarray(_HL),
      jnp.asarray(_G))


def _bcast_lane(v, j):
    dnums = lax.GatherDimensionNumbers(
        offset_dims=(), collapsed_slice_dims=(0,), start_index_map=(0,))
    return lax.gather(v, jnp.full((16, 1), j, jnp.int32), dnums, (1,),
                      mode=lax.GatherScatterMode.PROMISE_IN_BOUNDS)


_GRP = 16
_NGRP = _QPW // _GRP
_NPAIR = _QPW // 2


def _sc_body(val_hbm, idx_hbm, wgt_hbm, out_hbm, ibuf, wbuf, rb0, rb1, obuf,
             sem_iw, sem_g0, sem_g1, sem_o):
    wid = lax.axis_index("s") * 2 + lax.axis_index("c")
    q0 = wid * _QPW

    def iw_copies(gdst):
        s = (q0 + gdst * _GRP) * 4
        hi = pltpu.make_async_copy(idx_hbm.at[pl.ds(s, _GRP * 4)], ibuf.at[gdst & 1], sem_iw)
        hw = pltpu.make_async_copy(wgt_hbm.at[pl.ds(s, _GRP * 4)], wbuf.at[gdst & 1], sem_iw)
        return hi, hw

    def gather(t, rb, sem):
        gb = (t // _GRP) & 1
        sl = t % _GRP
        return [pltpu.make_async_copy(val_hbm.at[ibuf.at[gb, sl * 4 + c]], rb.at[c], sem)
                for c in range(4)]

    def compute(t, rb):
        gb = (t // _GRP) & 1
        sl = t % _GRP

        def per_head(h, _):
            a0 = []
            a1 = []
            for c in range(4):
                w16 = wbuf[gb, sl * 4 + c, pl.ds(h * 16, 16)]
                accb = None
                for j in range(16):
                    wj = _bcast_lane(w16, j)
                    wjb = plsc.pack(wj, wj, format=plsc.PackFormat.INTERLEAVED)
                    rbf = plsc.bitcast(rb[c, h * 16 + j], jnp.bfloat16)
                    t = wjb * rbf
                    accb = t if accb is None else accb + t
                lo, hi = plsc.unpack(accb, format=plsc.PackFormat.INTERLEAVED)
                a0.append(lo)
                a1.append(hi)
            obuf[gb, sl, pl.ds(h * 32, 16)] = (a0[0] + a0[1]) + (a0[2] + a0[3])
            obuf[gb, sl, pl.ds(h * 32 + 16, 16)] = (a1[0] + a1[1]) + (a1[2] + a1[3])
            return 0

        lax.fori_loop(0, _NH, per_head, 0)

    hi, hw = iw_copies(0)
    hi.start()
    hw.start()

    def pair(i2, _):
        g = i2 // (_GRP // 2)
        slot = i2 % (_GRP // 2)
        a = 2 * i2
        b = a + 1

        @pl.when(slot == 0)
        def _():
            h1, h2 = iw_copies(g)
            h1.wait()
            h2.wait()

            @pl.when(g + 1 < _NGRP)
            def _():
                h3, h4 = iw_copies(g + 1)
                h3.start()
                h4.start()

            @pl.when(g >= 2)
            def _():
                pltpu.make_async_copy(
                    obuf.at[g & 1],
                    out_hbm.at[pl.ds(q0 + (g - 2) * _GRP, _GRP)], sem_o).wait()

            for cp in gather(a, rb0, sem_g0):
                cp.start()

        for cp in gather(b, rb1, sem_g1):
            cp.start()
        for cp in gather(a, rb0, sem_g0):
            cp.wait()
        compute(a, rb0)

        @pl.when(slot < _GRP // 2 - 1)
        def _():
            for cp in gather(a + 2, rb0, sem_g0):
                cp.start()

        for cp in gather(b, rb1, sem_g1):
            cp.wait()
        compute(b, rb1)

        @pl.when(slot == _GRP // 2 - 1)
        def _():
            pltpu.make_async_copy(
                obuf.at[g & 1], out_hbm.at[pl.ds(q0 + g * _GRP, _GRP)], sem_o).start()
        return 0

    lax.fori_loop(0, _NPAIR, pair, 0)
    pltpu.make_async_copy(obuf.at[0], out_hbm.at[pl.ds(q0, _GRP)], sem_o).wait()
    pltpu.make_async_copy(obuf.at[0], out_hbm.at[pl.ds(q0, _GRP)], sem_o).wait()


@functools.cache
def _sc_gather():
    mesh = plsc.VectorSubcoreMesh(core_axis_name="c", subcore_axis_name="s")
    return pl.kernel(
        _sc_body,
        mesh=mesh,
        compiler_params=pltpu.CompilerParams(use_tc_tiling_on_sc=False,
                                             needs_layout_passes=False),
        out_type=jax.ShapeDtypeStruct((_NQ, _D), jnp.float32),
        scratch_types=[
            pltpu.VMEM((2, _GRP * 4, 128), jnp.int32),
            pltpu.VMEM((2, _GRP * 4, 128), jnp.float32),
            pltpu.VMEM((4, 128, _DH // 2), jnp.int32),
            pltpu.VMEM((4, 128, _DH // 2), jnp.int32),
            pltpu.VMEM((2, _GRP, _D), jnp.float32),
            pltpu.SemaphoreType.DMA,
            pltpu.SemaphoreType.DMA,
            pltpu.SemaphoreType.DMA,
            pltpu.SemaphoreType.DMA,
        ],
    )


def _ln_inline(x, g, b):
    m = jnp.mean(x, -1, keepdims=True)
    v = jnp.mean((x - m) ** 2, -1, keepdims=True)
    return (x - m) / jnp.sqrt(v + 1e-5) * g + b


def _kb_body(x_ref, at_ref, wo_ref, bo_ref, g1_ref, be1_ref, w1_ref, bf1_ref,
             w2_ref, bf2_ref, g2_ref, be2_ref, o_ref):
    x = x_ref[0]
    a = at_ref[0]
    h1 = x + jnp.dot(a, wo_ref[...], preferred_element_type=jnp.float32) + bo_ref[...]
    x1 = _ln_inline(h1, g1_ref[...], be1_ref[...])
    ff = jnp.maximum(jnp.dot(x1, w1_ref[...], preferred_element_type=jnp.float32) + bf1_ref[...], 0.0)
    ff2 = jnp.dot(ff, w2_ref[...], preferred_element_type=jnp.float32) + bf2_ref[...]
    o_ref[0] = _ln_inline(x1 + ff2, g2_ref[...], be2_ref[...])


def _kernel_b(x, attn, wo, bo, g1, be1, w1, bf1, w2, bf2, g2, be2, interpret=False):
    c1 = lambda i, j: (0, 0)
    return pl.pallas_call(
        _kb_body,
        grid=(_B, _NBLK),
        in_specs=[
            pl.BlockSpec((1, _CHUNK, _D), lambda i, j: (i, j, 0)),
            pl.BlockSpec((1, _CHUNK, _D), lambda i, j: (i, j, 0)),
            pl.BlockSpec((_D, _D), c1),
            pl.BlockSpec((1, _D), c1),
            pl.BlockSpec((1, _D), c1),
            pl.BlockSpec((1, _D), c1),
            pl.BlockSpec((_D, 1024), c1),
            pl.BlockSpec((1, 1024), c1),
            pl.BlockSpec((1024, _D), c1),
            pl.BlockSpec((1, _D), c1),
            pl.BlockSpec((1, _D), c1),
            pl.BlockSpec((1, _D), c1),
        ],
        out_specs=pl.BlockSpec((1, _CHUNK, _D), lambda i, j: (i, j, 0)),
        out_shape=jax.ShapeDtypeStruct((_B, _LQ, _D), jnp.float32),
        interpret=interpret,
    )(x, attn, wo, bo, g1, be1, w1, bf1, w2, bf2, g2, be2)


def kernel(src0, src1, src2, src3, pos0, pos1, pos2, pos3, level_embed,
           Wso, bso, Waw, baw, Wv, bv, Wo, bo, g1, be1, W1, bf1, W2, bf2, g2, be2):
    srcs = [src0, src1, src2, src3]
    poss = [pos0, pos1, pos2, pos3]
    x = jnp.concatenate([s.reshape(_B, _D, -1).transpose(0, 2, 1) for s in srcs], 1)
    qpos = jnp.concatenate(
        [p.reshape(_B, _D, -1).transpose(0, 2, 1) + level_embed[i][None, None, :]
         for i, p in enumerate(poss)], 1)

    perm = jnp.asarray(_PERM)
    permv = jnp.asarray(_PERM_V)
    refx = jnp.asarray(_REFX)
    refy = jnp.asarray(_REFY)
    sc = _sc_gather()

    for li in range(_NLAYERS):
        wso_p = Wso[li][:, perm]
        bso_p = bso[li][perm][None]
        value, idx, wgt = _kernel_a(x, qpos, refx, refy, wso_p, bso_p,
                                    Waw[li], baw[li][None],
                                    Wv[li][:, permv], bv[li][permv][None])
        attn = sc(value.reshape(_NQ * _NH, _DH // 2),
                  idx.reshape(_NQ * 4, 128),
                  wgt.reshape(_NQ * 4, 128))
        x = _kernel_b(x, attn.reshape(_B, _LQ, _D), Wo[li], bo[li][None],
                      g1[li][None], be1[li][None], W1[li], bf1[li][None],
                      W2[li], bf2[li][None], g2[li][None], be2[li][None])
    return x


# speedup vs baseline: 5.8998x; 1.0080x over previous
import functools

import jax
import jax.numpy as jnp
import numpy as np
from jax import lax
from jax.experimental import pallas as pl
from jax.experimental.pallas import tpu as pltpu
from jax.experimental.pallas import tpu_sc as plsc

_SHAPES = [(128, 128), (64, 64), (32, 32), (16, 16)]
_B = 2
_D = 256
_NLAYERS = 3
_NH = 8
_NL = 4
_NP = 4
_DH = _D // _NH
_LQ = sum(h * w for h, w in _SHAPES)
_CHUNK = 1280
_NBLK = _LQ // _CHUNK
_NQ = _B * _LQ
_NWORK = 32
_QPW = _NQ // _NWORK

_lane = np.arange(128)
_h_of = _lane // 16
_l_of = (_lane // 4) % 4
_W_LVL = np.array([w for h, w in _SHAPES], np.float32)
_H_LVL = np.array([h for h, w in _SHAPES], np.float32)
_BASE_LVL = np.cumsum([0] + [h * w for h, w in _SHAPES])[:4]

_SW = _W_LVL[_l_of].reshape(1, 128)
_SH = _H_LVL[_l_of].reshape(1, 128)
_BASE = _BASE_LVL[_l_of].astype(np.int32).reshape(1, 128)
_HL = _h_of.astype(np.int32).reshape(1, 128)

_G = np.kron(np.eye(8, dtype=np.float32), np.ones((16, 16), np.float32))

_PERM = np.empty(256, np.int32)
for _c in range(2):
    for _hh in range(8):
        for _ll in range(4):
            for _pp in range(4):
                _PERM[_c * 128 + _hh * 16 + _ll * 4 + _pp] = ((_hh * 4 + _ll) * 4 + _pp) * 2 + _c

_PERM_V = np.empty(256, np.int32)
for _hh in range(8):
    for _k in range(16):
        _PERM_V[_hh * 16 + _k] = _hh * 32 + _k
        _PERM_V[128 + _hh * 16 + _k] = _hh * 32 + 16 + _k


def _ref_points_np():
    xs, ys = [], []
    for h, w in _SHAPES:
        ry = (np.arange(h, dtype=np.float32) + 0.5) / h
        rx = (np.arange(w, dtype=np.float32) + 0.5) / w
        gy, gx = np.meshgrid(ry, rx, indexing="ij")
        xs.append(gx.reshape(-1))
        ys.append(gy.reshape(-1))
    return np.concatenate(xs), np.concatenate(ys)


_REFX, _REFY = _ref_points_np()
_REFX = _REFX.reshape(_LQ, 1)
_REFY = _REFY.reshape(_LQ, 1)


def _ka_body(x_ref, qp_ref, refx_ref, refy_ref, wso_ref, bso_ref, waw_ref,
             baw_ref, wv_ref, bv_ref, sw_ref, sh_ref, base_ref, hl_ref, g_ref,
             val_ref, idx_ref, wgt_ref):
    b = pl.program_id(0)
    x = x_ref[0]
    q = x + qp_ref[0]
    off = jnp.dot(q, wso_ref[...], preferred_element_type=jnp.float32) + bso_ref[...]
    logits = jnp.dot(q, waw_ref[...], preferred_element_type=jnp.float32) + baw_ref[...]
    m = jnp.max(logits, axis=-1, keepdims=True)
    e = jnp.exp(logits - m)
    gs = lax.dot(e, g_ref[...], precision=lax.Precision.HIGHEST)
    aw = e / gs
    v = jnp.dot(x, wv_ref[...], preferred_element_type=jnp.float32) + bv_ref[...]
    u = lax.bitcast_convert_type(v, jnp.int32)
    r = u + jnp.int32(0x7FFF) + ((u >> 16) & 1)
    lo = (r[:, :128] >> 16) & jnp.int32(0xFFFF)
    hi = r[:, 128:] & jnp.int32(-65536)
    val_ref[0] = lo | hi

    offx = off[:, :128]
    offy = off[:, 128:]
    sw = sw_ref[...]
    sh = sh_ref[...]
    xi = refx_ref[...] * sw + offx - 0.5
    yi = refy_ref[...] * sh + offy - 0.5
    x0 = jnp.floor(xi)
    y0 = jnp.floor(yi)
    fx = xi - x0
    fy = yi - y0
    x0i = x0.astype(jnp.int32)
    y0i = y0.astype(jnp.int32)
    swi = sw.astype(jnp.int32)
    shi = sh.astype(jnp.int32)
    rowbase = b * _LQ + base_ref[...]
    hl = hl_ref[...]

    outs_i = []
    outs_w = []
    for dy in (0, 1):
        wy = (1.0 - fy) if dy == 0 else fy
        yc = y0i + dy
        vy = (yc >= 0) & (yc <= shi - 1)
        cy = jnp.clip(yc, 0, shi - 1)
        for dx in (0, 1):
            wx = (1.0 - fx) if dx == 0 else fx
            xc = x0i + dx
            vx = (xc >= 0) & (xc <= swi - 1)
            cx = jnp.clip(xc, 0, swi - 1)
            wc = wx * wy * (vx & vy).astype(jnp.float32) * aw
            row = (rowbase + cy * swi + cx) * _NH + hl
            outs_i.append(row)
            wu = lax.bitcast_convert_type(wc, jnp.int32)
            wr = wu + jnp.int32(0x7FFF) + ((wu >> 16) & 1)
            wh = wr & jnp.int32(-65536)
            outs_w.append(wh | ((wh >> 16) & jnp.int32(0xFFFF)))
    idx_ref[0] = jnp.concatenate(outs_i, axis=-1)
    wgt_ref[0] = jnp.concatenate(outs_w, axis=-1)


def _kernel_a(x, qpos, refx, refy, wso, bso, waw, baw, wv, bv, interpret=False):
    c1 = lambda i, j: (0, 0)
    return pl.pallas_call(
        _ka_body,
        grid=(_B, _NBLK),
        in_specs=[
            pl.BlockSpec((1, _CHUNK, _D), lambda i, j: (i, j, 0)),
            pl.BlockSpec((1, _CHUNK, _D), lambda i, j: (i, j, 0)),
            pl.BlockSpec((_CHUNK, 1), lambda i, j: (j, 0)),
            pl.BlockSpec((_CHUNK, 1), lambda i, j: (j, 0)),
            pl.BlockSpec((_D, 256), c1),
            pl.BlockSpec((1, 256), c1),
            pl.BlockSpec((_D, 128), c1),
            pl.BlockSpec((1, 128), c1),
            pl.BlockSpec((_D, _D), c1),
            pl.BlockSpec((1, _D), c1),
            pl.BlockSpec((1, 128), c1),
            pl.BlockSpec((1, 128), c1),
            pl.BlockSpec((1, 128), c1),
            pl.BlockSpec((1, 128), c1),
            pl.BlockSpec((128, 128), c1),
        ],
        out_specs=[
            pl.BlockSpec((1, _CHUNK, 128), lambda i, j: (i, j, 0)),
            pl.BlockSpec((1, _CHUNK, 512), lambda i, j: (i, j, 0)),
            pl.BlockSpec((1, _CHUNK, 512), lambda i, j: (i, j, 0)),
        ],
        out_shape=[
            jax.ShapeDtypeStruct((_B, _LQ, 128), jnp.int32),
            jax.ShapeDtypeStruct((_B, _LQ, 512), jnp.int32),
            jax.ShapeDtypeStruct((_B, _LQ, 512), jnp.int32),
        ],
        interpret=interpret,
    )(x, qpos, refx, refy, wso, bso, waw, baw, wv, bv,
      jnp.asarray(_SW), jnp.asarray(_SH), jnp.asarray(_BASE), jnp.asarray(_HL),
      jnp.asarray(_G))


def _bcast_lane(v, j):
    dnums = lax.GatherDimensionNumbers(
        offset_dims=(), collapsed_slice_dims=(0,), start_index_map=(0,))
    return lax.gather(v, jnp.full((16, 1), j, jnp.int32), dnums, (1,),
                      mode=lax.GatherScatterMode.PROMISE_IN_BOUNDS)


_GRP = 16
_NGRP = _QPW // _GRP
_NPAIR = _QPW // 2


def _sc_body(val_hbm, idx_hbm, wgt_hbm, out_hbm, ibuf, wbuf, rb0, rb1, obuf,
             sem_iw, sem_g0, sem_g1, sem_o):
    wid = lax.axis_index("s") * 2 + lax.axis_index("c")
    q0 = wid * _QPW

    def iw_copies(gdst):
        s = (q0 + gdst * _GRP) * 4
        hi = pltpu.make_async_copy(idx_hbm.at[pl.ds(s, _GRP * 4)], ibuf.at[gdst & 1], sem_iw)
        hw = pltpu.make_async_copy(wgt_hbm.at[pl.ds(s, _GRP * 4)], wbuf.at[gdst & 1], sem_iw)
        return hi, hw

    def gather(t, rb, sem):
        gb = (t // _GRP) & 1
        sl = t % _GRP
        return [pltpu.make_async_copy(val_hbm.at[ibuf.at[gb, sl * 4 + c]], rb.at[c], sem)
                for c in range(4)]

    def compute(t, rb):
        gb = (t // _GRP) & 1
        sl = t % _GRP

        def per_head(h, _):
            a0 = []
            a1 = []
            for c in range(4):
                w16 = wbuf[gb, sl * 4 + c, pl.ds(h * 16, 16)]
                accb = None
                for j in range(16):
                    wjb = plsc.bitcast(_bcast_lane(w16, j), jnp.bfloat16)
                    rbf = plsc.bitcast(rb[c, h * 16 + j], jnp.bfloat16)
                    t = wjb * rbf
                    accb = t if accb is None else accb + t
                lo, hi = plsc.unpack(accb, format=plsc.PackFormat.INTERLEAVED)
                a0.append(lo)
                a1.append(hi)
            obuf[gb, sl, pl.ds(h * 32, 16)] = (a0[0] + a0[1]) + (a0[2] + a0[3])
            obuf[gb, sl, pl.ds(h * 32 + 16, 16)] = (a1[0] + a1[1]) + (a1[2] + a1[3])
            return 0

        lax.fori_loop(0, _NH, per_head, 0)

    hi, hw = iw_copies(0)
    hi.start()
    hw.start()

    def pair(i2, _):
        g = i2 // (_GRP // 2)
        slot = i2 % (_GRP // 2)
        a = 2 * i2
        b = a + 1

        @pl.when(slot == 0)
        def _():
            h1, h2 = iw_copies(g)
            h1.wait()
            h2.wait()

            @pl.when(g + 1 < _NGRP)
            def _():
                h3, h4 = iw_copies(g + 1)
                h3.start()
                h4.start()

            @pl.when(g >= 2)
            def _():
                pltpu.make_async_copy(
                    obuf.at[g & 1],
                    out_hbm.at[pl.ds(q0 + (g - 2) * _GRP, _GRP)], sem_o).wait()

            for cp in gather(a, rb0, sem_g0):
                cp.start()

        for cp in gather(b, rb1, sem_g1):
            cp.start()
        for cp in gather(a, rb0, sem_g0):
            cp.wait()
        compute(a, rb0)

        @pl.when(slot < _GRP // 2 - 1)
        def _():
            for cp in gather(a + 2, rb0, sem_g0):
                cp.start()

        for cp in gather(b, rb1, sem_g1):
            cp.wait()
        compute(b, rb1)

        @pl.when(slot == _GRP // 2 - 1)
        def _():
            pltpu.make_async_copy(
                obuf.at[g & 1], out_hbm.at[pl.ds(q0 + g * _GRP, _GRP)], sem_o).start()
        return 0

    lax.fori_loop(0, _NPAIR, pair, 0)
    pltpu.make_async_copy(obuf.at[0], out_hbm.at[pl.ds(q0, _GRP)], sem_o).wait()
    pltpu.make_async_copy(obuf.at[0], out_hbm.at[pl.ds(q0, _GRP)], sem_o).wait()


@functools.cache
def _sc_gather():
    mesh = plsc.VectorSubcoreMesh(core_axis_name="c", subcore_axis_name="s")
    return pl.kernel(
        _sc_body,
        mesh=mesh,
        compiler_params=pltpu.CompilerParams(use_tc_tiling_on_sc=False,
                                             needs_layout_passes=False),
        out_type=jax.ShapeDtypeStruct((_NQ, _D), jnp.float32),
        scratch_types=[
            pltpu.VMEM((2, _GRP * 4, 128), jnp.int32),
            pltpu.VMEM((2, _GRP * 4, 128), jnp.int32),
            pltpu.VMEM((4, 128, _DH // 2), jnp.int32),
            pltpu.VMEM((4, 128, _DH // 2), jnp.int32),
            pltpu.VMEM((2, _GRP, _D), jnp.float32),
            pltpu.SemaphoreType.DMA,
            pltpu.SemaphoreType.DMA,
            pltpu.SemaphoreType.DMA,
            pltpu.SemaphoreType.DMA,
        ],
    )


def _ln_inline(x, g, b):
    m = jnp.mean(x, -1, keepdims=True)
    v = jnp.mean((x - m) ** 2, -1, keepdims=True)
    return (x - m) / jnp.sqrt(v + 1e-5) * g + b


def _kb_body(x_ref, at_ref, wo_ref, bo_ref, g1_ref, be1_ref, w1_ref, bf1_ref,
             w2_ref, bf2_ref, g2_ref, be2_ref, o_ref):
    x = x_ref[0]
    a = at_ref[0]
    h1 = x + jnp.dot(a, wo_ref[...], preferred_element_type=jnp.float32) + bo_ref[...]
    x1 = _ln_inline(h1, g1_ref[...], be1_ref[...])
    ff = jnp.maximum(jnp.dot(x1, w1_ref[...], preferred_element_type=jnp.float32) + bf1_ref[...], 0.0)
    ff2 = jnp.dot(ff, w2_ref[...], preferred_element_type=jnp.float32) + bf2_ref[...]
    o_ref[0] = _ln_inline(x1 + ff2, g2_ref[...], be2_ref[...])


def _kernel_b(x, attn, wo, bo, g1, be1, w1, bf1, w2, bf2, g2, be2, interpret=False):
    c1 = lambda i, j: (0, 0)
    return pl.pallas_call(
        _kb_body,
        grid=(_B, _NBLK),
        in_specs=[
            pl.BlockSpec((1, _CHUNK, _D), lambda i, j: (i, j, 0)),
            pl.BlockSpec((1, _CHUNK, _D), lambda i, j: (i, j, 0)),
            pl.BlockSpec((_D, _D), c1),
            pl.BlockSpec((1, _D), c1),
            pl.BlockSpec((1, _D), c1),
            pl.BlockSpec((1, _D), c1),
            pl.BlockSpec((_D, 1024), c1),
            pl.BlockSpec((1, 1024), c1),
            pl.BlockSpec((1024, _D), c1),
            pl.BlockSpec((1, _D), c1),
            pl.BlockSpec((1, _D), c1),
            pl.BlockSpec((1, _D), c1),
        ],
        out_specs=pl.BlockSpec((1, _CHUNK, _D), lambda i, j: (i, j, 0)),
        out_shape=jax.ShapeDtypeStruct((_B, _LQ, _D), jnp.float32),
        interpret=interpret,
    )(x, attn, wo, bo, g1, be1, w1, bf1, w2, bf2, g2, be2)


def kernel(src0, src1, src2, src3, pos0, pos1, pos2, pos3, level_embed,
           Wso, bso, Waw, baw, Wv, bv, Wo, bo, g1, be1, W1, bf1, W2, bf2, g2, be2):
    srcs = [src0, src1, src2, src3]
    poss = [pos0, pos1, pos2, pos3]
    x = jnp.concatenate([s.reshape(_B, _D, -1).transpose(0, 2, 1) for s in srcs], 1)
    qpos = jnp.concatenate(
        [p.reshape(_B, _D, -1).transpose(0, 2, 1) + level_embed[i][None, None, :]
         for i, p in enumerate(poss)], 1)

    perm = jnp.asarray(_PERM)
    permv = jnp.asarray(_PERM_V)
    refx = jnp.asarray(_REFX)
    refy = jnp.asarray(_REFY)
    sc = _sc_gather()

    for li in range(_NLAYERS):
        wso_p = Wso[li][:, perm]
        bso_p = bso[li][perm][None]
        value, idx, wgt = _kernel_a(x, qpos, refx, refy, wso_p, bso_p,
                                    Waw[li], baw[li][None],
                                    Wv[li][:, permv], bv[li][permv][None])
        attn = sc(value.reshape(_NQ * _NH, _DH // 2),
                  idx.reshape(_NQ * 4, 128),
                  wgt.reshape(_NQ * 4, 128))
        x = _kernel_b(x, attn.reshape(_B, _LQ, _D), Wo[li], bo[li][None],
                      g1[li][None], be1[li][None], W1[li], bf1[li][None],
                      W2[li], bf2[li][None], g2[li][None], be2[li][None])
    return x


# speedup vs baseline: 6.1607x; 1.0442x over previous
import functools

import jax
import jax.numpy as jnp
import numpy as np
from jax import lax
from jax.experimental import pallas as pl
from jax.experimental.pallas import tpu as pltpu
from jax.experimental.pallas import tpu_sc as plsc

_SHAPES = [(128, 128), (64, 64), (32, 32), (16, 16)]
_B = 2
_D = 256
_NLAYERS = 3
_NH = 8
_NL = 4
_NP = 4
_DH = _D // _NH
_LQ = sum(h * w for h, w in _SHAPES)
_CHUNK = 1280
_NBLK = _LQ // _CHUNK
_NQ = _B * _LQ
_NWORK = 32
_QPW = _NQ // _NWORK

_lane = np.arange(128)
_h_of = _lane // 16
_l_of = (_lane // 4) % 4
_W_LVL = np.array([w for h, w in _SHAPES], np.float32)
_H_LVL = np.array([h for h, w in _SHAPES], np.float32)
_BASE_LVL = np.cumsum([0] + [h * w for h, w in _SHAPES])[:4]

_SW = _W_LVL[_l_of].reshape(1, 128)
_SH = _H_LVL[_l_of].reshape(1, 128)
_BASE = _BASE_LVL[_l_of].astype(np.int32).reshape(1, 128)
_HL = _h_of.astype(np.int32).reshape(1, 128)

_G = np.kron(np.eye(8, dtype=np.float32), np.ones((16, 16), np.float32))

_PERM = np.empty(256, np.int32)
for _c in range(2):
    for _hh in range(8):
        for _ll in range(4):
            for _pp in range(4):
                _PERM[_c * 128 + _hh * 16 + _ll * 4 + _pp] = ((_hh * 4 + _ll) * 4 + _pp) * 2 + _c

_PERM_V = np.empty(256, np.int32)
for _hh in range(8):
    for _k in range(16):
        _PERM_V[_hh * 16 + _k] = _hh * 32 + _k
        _PERM_V[128 + _hh * 16 + _k] = _hh * 32 + 16 + _k


def _ref_points_np():
    xs, ys = [], []
    for h, w in _SHAPES:
        ry = (np.arange(h, dtype=np.float32) + 0.5) / h
        rx = (np.arange(w, dtype=np.float32) + 0.5) / w
        gy, gx = np.meshgrid(ry, rx, indexing="ij")
        xs.append(gx.reshape(-1))
        ys.append(gy.reshape(-1))
    return np.concatenate(xs), np.concatenate(ys)


_REFX, _REFY = _ref_points_np()
_REFX = _REFX.reshape(_LQ, 1)
_REFY = _REFY.reshape(_LQ, 1)


def _ka_body(x_ref, qp_ref, refx_ref, refy_ref, wso_ref, bso_ref, waw_ref,
             baw_ref, wv_ref, bv_ref, sw_ref, sh_ref, base_ref, hl_ref, g_ref,
             val_ref, idx_ref, wgt_ref):
    b = pl.program_id(0)
    x = x_ref[0]
    q = x + qp_ref[0]
    off = jnp.dot(q, wso_ref[...], preferred_element_type=jnp.float32) + bso_ref[...]
    logits = jnp.dot(q, waw_ref[...], preferred_element_type=jnp.float32) + baw_ref[...]
    m = jnp.max(logits, axis=-1, keepdims=True)
    e = jnp.exp(logits - m)
    gs = lax.dot(e, g_ref[...], precision=lax.Precision.HIGHEST)
    aw = e / gs
    v = jnp.dot(x, wv_ref[...], preferred_element_type=jnp.float32) + bv_ref[...]
    u = lax.bitcast_convert_type(v, jnp.int32)
    r = u + jnp.int32(0x7FFF) + ((u >> 16) & 1)
    lo = (r[:, :128] >> 16) & jnp.int32(0xFFFF)
    hi = r[:, 128:] & jnp.int32(-65536)
    val_ref[0] = lo | hi

    offx = off[:, :128]
    offy = off[:, 128:]
    sw = sw_ref[...]
    sh = sh_ref[...]
    xi = refx_ref[...] * sw + offx - 0.5
    yi = refy_ref[...] * sh + offy - 0.5
    x0 = jnp.floor(xi)
    y0 = jnp.floor(yi)
    fx = xi - x0
    fy = yi - y0
    x0i = x0.astype(jnp.int32)
    y0i = y0.astype(jnp.int32)
    swi = sw.astype(jnp.int32)
    shi = sh.astype(jnp.int32)
    rowbase = b * _LQ + base_ref[...]
    hl = hl_ref[...]

    outs_i = []
    outs_w = []
    for dy in (0, 1):
        wy = (1.0 - fy) if dy == 0 else fy
        yc = y0i + dy
        vy = (yc >= 0) & (yc <= shi - 1)
        cy = jnp.clip(yc, 0, shi - 1)
        for dx in (0, 1):
            wx = (1.0 - fx) if dx == 0 else fx
            xc = x0i + dx
            vx = (xc >= 0) & (xc <= swi - 1)
            cx = jnp.clip(xc, 0, swi - 1)
            wc = wx * wy * (vx & vy).astype(jnp.float32) * aw
            row = (rowbase + cy * swi + cx) * _NH + hl
            outs_i.append(row)
            wu = lax.bitcast_convert_type(wc, jnp.int32)
            wr = wu + jnp.int32(0x7FFF) + ((wu >> 16) & 1)
            wh = wr & jnp.int32(-65536)
            outs_w.append(wh | ((wh >> 16) & jnp.int32(0xFFFF)))
    idx_ref[0] = jnp.concatenate(outs_i, axis=-1)
    wgt_ref[0] = jnp.concatenate(outs_w, axis=-1)


def _kernel_a(x, qpos, refx, refy, wso, bso, waw, baw, wv, bv, interpret=False):
    c1 = lambda i, j: (0, 0)
    return pl.pallas_call(
        _ka_body,
        grid=(_B, _NBLK),
        in_specs=[
            pl.BlockSpec((1, _CHUNK, _D), lambda i, j: (i, j, 0)),
            pl.BlockSpec((1, _CHUNK, _D), lambda i, j: (i, j, 0)),
            pl.BlockSpec((_CHUNK, 1), lambda i, j: (j, 0)),
            pl.BlockSpec((_CHUNK, 1), lambda i, j: (j, 0)),
            pl.BlockSpec((_D, 256), c1),
            pl.BlockSpec((1, 256), c1),
            pl.BlockSpec((_D, 128), c1),
            pl.BlockSpec((1, 128), c1),
            pl.BlockSpec((_D, _D), c1),
            pl.BlockSpec((1, _D), c1),
            pl.BlockSpec((1, 128), c1),
            pl.BlockSpec((1, 128), c1),
            pl.BlockSpec((1, 128), c1),
            pl.BlockSpec((1, 128), c1),
            pl.BlockSpec((128, 128), c1),
        ],
        out_specs=[
            pl.BlockSpec((1, _CHUNK, 128), lambda i, j: (i, j, 0)),
            pl.BlockSpec((1, _CHUNK, 512), lambda i, j: (i, j, 0)),
            pl.BlockSpec((1, _CHUNK, 512), lambda i, j: (i, j, 0)),
        ],
        out_shape=[
            jax.ShapeDtypeStruct((_B, _LQ, 128), jnp.int32),
            jax.ShapeDtypeStruct((_B, _LQ, 512), jnp.int32),
            jax.ShapeDtypeStruct((_B, _LQ, 512), jnp.int32),
        ],
        interpret=interpret,
    )(x, qpos, refx, refy, wso, bso, waw, baw, wv, bv,
      jnp.asarray(_SW), jnp.asarray(_SH), jnp.asarray(_BASE), jnp.asarray(_HL),
      jnp.asarray(_G))


def _bcast_lane(v, j):
    dnums = lax.GatherDimensionNumbers(
        offset_dims=(), collapsed_slice_dims=(0,), start_index_map=(0,))
    return lax.gather(v, jnp.full((16, 1), j, jnp.int32), dnums, (1,),
                      mode=lax.GatherScatterMode.PROMISE_IN_BOUNDS)


_GRP = 16
_NGRP = _QPW // _GRP
_NPAIR = _QPW // 2


def _sc_body(val_hbm, idx_hbm, wgt_hbm, out_hbm, ibuf, wbuf, rb0, rb1, obuf,
             sem_iw, sem_g0, sem_g1, sem_o):
    wid = lax.axis_index("s") * 2 + lax.axis_index("c")
    q0 = wid * _QPW

    def iw_copies(gdst):
        s = (q0 + gdst * _GRP) * 4
        hi = pltpu.make_async_copy(idx_hbm.at[pl.ds(s, _GRP * 4)], ibuf.at[gdst & 1], sem_iw)
        hw = pltpu.make_async_copy(wgt_hbm.at[pl.ds(s, _GRP * 4)], wbuf.at[gdst & 1], sem_iw)
        return hi, hw

    def gather(t, rb, sem):
        gb = (t // _GRP) & 1
        sl = t % _GRP
        return [pltpu.make_async_copy(val_hbm.at[ibuf.at[gb, sl * 4 + c]], rb.at[c], sem)
                for c in range(4)]

    def compute(t, rb):
        gb = (t // _GRP) & 1
        sl = t % _GRP

        def per_head(h, _):
            a0 = []
            a1 = []
            for c in range(4):
                w16 = wbuf[gb, sl * 4 + c, pl.ds(h * 16, 16)]
                accb = None
                for j in range(16):
                    wjb = plsc.bitcast(_bcast_lane(w16, j), jnp.bfloat16)
                    rbf = plsc.bitcast(rb[c, h * 16 + j], jnp.bfloat16)
                    t = wjb * rbf
                    accb = t if accb is None else accb + t
                lo, hi = plsc.unpack(accb, format=plsc.PackFormat.INTERLEAVED)
                a0.append(lo)
                a1.append(hi)
            obuf[gb, sl, pl.ds(h * 32, 16)] = (a0[0] + a0[1]) + (a0[2] + a0[3])
            obuf[gb, sl, pl.ds(h * 32 + 16, 16)] = (a1[0] + a1[1]) + (a1[2] + a1[3])
            return 0

        lax.fori_loop(0, _NH, per_head, 0)

    h1, h2 = iw_copies(0)
    h1.start()
    h2.start()
    h1.wait()
    h2.wait()
    h3, h4 = iw_copies(1)
    h3.start()
    h4.start()
    for cp in gather(0, rb0, sem_g0):
        cp.start()

    def pair(i2, _):
        g = i2 // (_GRP // 2)
        slot = i2 % (_GRP // 2)
        a = 2 * i2
        b = a + 1

        @pl.when((slot == 0) & (g >= 1) & (g + 1 < _NGRP))
        def _():
            h5, h6 = iw_copies(g + 1)
            h5.start()
            h6.start()

        @pl.when((slot == 0) & (g >= 2))
        def _():
            pltpu.make_async_copy(
                obuf.at[g & 1],
                out_hbm.at[pl.ds(q0 + (g - 2) * _GRP, _GRP)], sem_o).wait()

        for cp in gather(b, rb1, sem_g1):
            cp.start()
        for cp in gather(a, rb0, sem_g0):
            cp.wait()
        compute(a, rb0)

        @pl.when(slot < _GRP // 2 - 1)
        def _():
            for cp in gather(a + 2, rb0, sem_g0):
                cp.start()

        @pl.when((slot == _GRP // 2 - 1) & (g + 1 < _NGRP))
        def _():
            h7, h8 = iw_copies(g + 1)
            h7.wait()
            h8.wait()
            for cp in gather(a + 2, rb0, sem_g0):
                cp.start()

        for cp in gather(b, rb1, sem_g1):
            cp.wait()
        compute(b, rb1)

        @pl.when(slot == _GRP // 2 - 1)
        def _():
            pltpu.make_async_copy(
                obuf.at[g & 1], out_hbm.at[pl.ds(q0 + g * _GRP, _GRP)], sem_o).start()
        return 0

    lax.fori_loop(0, _NPAIR, pair, 0)
    pltpu.make_async_copy(obuf.at[0], out_hbm.at[pl.ds(q0, _GRP)], sem_o).wait()
    pltpu.make_async_copy(obuf.at[0], out_hbm.at[pl.ds(q0, _GRP)], sem_o).wait()


@functools.cache
def _sc_gather():
    mesh = plsc.VectorSubcoreMesh(core_axis_name="c", subcore_axis_name="s")
    return pl.kernel(
        _sc_body,
        mesh=mesh,
        compiler_params=pltpu.CompilerParams(use_tc_tiling_on_sc=False,
                                             needs_layout_passes=False),
        out_type=jax.ShapeDtypeStruct((_NQ, _D), jnp.float32),
        scratch_types=[
            pltpu.VMEM((2, _GRP * 4, 128), jnp.int32),
            pltpu.VMEM((2, _GRP * 4, 128), jnp.int32),
            pltpu.VMEM((4, 128, _DH // 2), jnp.int32),
            pltpu.VMEM((4, 128, _DH // 2), jnp.int32),
            pltpu.VMEM((2, _GRP, _D), jnp.float32),
            pltpu.SemaphoreType.DMA,
            pltpu.SemaphoreType.DMA,
            pltpu.SemaphoreType.DMA,
            pltpu.SemaphoreType.DMA,
        ],
    )


def _ln_inline(x, g, b):
    m = jnp.mean(x, -1, keepdims=True)
    v = jnp.mean((x - m) ** 2, -1, keepdims=True)
    return (x - m) / jnp.sqrt(v + 1e-5) * g + b


def _kb_body(x_ref, at_ref, wo_ref, bo_ref, g1_ref, be1_ref, w1_ref, bf1_ref,
             w2_ref, bf2_ref, g2_ref, be2_ref, o_ref):
    x = x_ref[0]
    a = at_ref[0]
    h1 = x + jnp.dot(a, wo_ref[...], preferred_element_type=jnp.float32) + bo_ref[...]
    x1 = _ln_inline(h1, g1_ref[...], be1_ref[...])
    ff = jnp.maximum(jnp.dot(x1, w1_ref[...], preferred_element_type=jnp.float32) + bf1_ref[...], 0.0)
    ff2 = jnp.dot(ff, w2_ref[...], preferred_element_type=jnp.float32) + bf2_ref[...]
    o_ref[0] = _ln_inline(x1 + ff2, g2_ref[...], be2_ref[...])


def _kernel_b(x, attn, wo, bo, g1, be1, w1, bf1, w2, bf2, g2, be2, interpret=False):
    c1 = lambda i, j: (0, 0)
    return pl.pallas_call(
        _kb_body,
        grid=(_B, _NBLK),
        in_specs=[
            pl.BlockSpec((1, _CHUNK, _D), lambda i, j: (i, j, 0)),
            pl.BlockSpec((1, _CHUNK, _D), lambda i, j: (i, j, 0)),
            pl.BlockSpec((_D, _D), c1),
            pl.BlockSpec((1, _D), c1),
            pl.BlockSpec((1, _D), c1),
            pl.BlockSpec((1, _D), c1),
            pl.BlockSpec((_D, 1024), c1),
            pl.BlockSpec((1, 1024), c1),
            pl.BlockSpec((1024, _D), c1),
            pl.BlockSpec((1, _D), c1),
            pl.BlockSpec((1, _D), c1),
            pl.BlockSpec((1, _D), c1),
        ],
        out_specs=pl.BlockSpec((1, _CHUNK, _D), lambda i, j: (i, j, 0)),
        out_shape=jax.ShapeDtypeStruct((_B, _LQ, _D), jnp.float32),
        interpret=interpret,
    )(x, attn, wo, bo, g1, be1, w1, bf1, w2, bf2, g2, be2)


def kernel(src0, src1, src2, src3, pos0, pos1, pos2, pos3, level_embed,
           Wso, bso, Waw, baw, Wv, bv, Wo, bo, g1, be1, W1, bf1, W2, bf2, g2, be2):
    srcs = [src0, src1, src2, src3]
    poss = [pos0, pos1, pos2, pos3]
    x = jnp.concatenate([s.reshape(_B, _D, -1).transpose(0, 2, 1) for s in srcs], 1)
    qpos = jnp.concatenate(
        [p.reshape(_B, _D, -1).transpose(0, 2, 1) + level_embed[i][None, None, :]
         for i, p in enumerate(poss)], 1)

    perm = jnp.asarray(_PERM)
    permv = jnp.asarray(_PERM_V)
    refx = jnp.asarray(_REFX)
    refy = jnp.asarray(_REFY)
    sc = _sc_gather()

    for li in range(_NLAYERS):
        wso_p = Wso[li][:, perm]
        bso_p = bso[li][perm][None]
        value, idx, wgt = _kernel_a(x, qpos, refx, refy, wso_p, bso_p,
                                    Waw[li], baw[li][None],
                                    Wv[li][:, permv], bv[li][permv][None])
        attn = sc(value.reshape(_NQ * _NH, _DH // 2),
                  idx.reshape(_NQ * 4, 128),
                  wgt.reshape(_NQ * 4, 128))
        x = _kernel_b(x, attn.reshape(_B, _LQ, _D), Wo[li], bo[li][None],
                      g1[li][None], be1[li][None], W1[li], bf1[li][None],
                      W2[li], bf2[li][None], g2[li][None], be2[li][None])
    return x
